# Initial kernel scaffold; baseline (speedup 1.0000x reference)
import jax
import jax.numpy as jnp
from jax import lax
from jax.experimental import pallas as pl
from jax.experimental.pallas import tpu as pltpu

NPOINT = 2048
N = 65536
B = 8


def _fps_body(x_ref, y_ref, z_ref, out_ref, dists_ref):
    dists_ref[...] = jnp.full((1, N), 1e10, dtype=jnp.float32)
    x = x_ref[...]
    y = y_ref[...]
    z = z_ref[...]

    def it(i, farthest):
        out_ref[0, pl.ds(i, 1)] = jnp.full((1,), farthest, dtype=jnp.int32)
        cx = x_ref[0, pl.ds(farthest, 1)][0]
        cy = y_ref[0, pl.ds(farthest, 1)][0]
        cz = z_ref[0, pl.ds(farthest, 1)][0]
        dx = x - cx
        dy = y - cy
        dz = z - cz
        d = dx * dx + dy * dy + dz * dz
        nd = jnp.minimum(dists_ref[...], d)
        dists_ref[...] = nd
        return jnp.argmax(nd, axis=1)[0].astype(jnp.int32)

    lax.fori_loop(0, NPOINT, it, jnp.int32(0))


def kernel(pts):
    ptsT = jnp.transpose(pts, (2, 0, 1))
    x, y, z = ptsT[0], ptsT[1], ptsT[2]
    spec = pl.BlockSpec((1, N), lambda b: (b, 0))
    out = pl.pallas_call(
        _fps_body,
        grid=(B,),
        in_specs=[spec, spec, spec],
        out_specs=pl.BlockSpec((1, NPOINT), lambda b: (b, 0)),
        out_shape=jax.ShapeDtypeStruct((B, NPOINT), jnp.int32),
        scratch_shapes=[pltpu.VMEM((1, N), jnp.float32)],
    )(x, y, z)
    return out


# speedup vs baseline: 8.0005x; 8.0005x over previous
import jax
import jax.numpy as jnp
from jax import lax
from jax.experimental import pallas as pl
from jax.experimental.pallas import tpu as pltpu

NPOINT = 2048
N = 65536
B = 8
CHUNK = 512


def _fps_body(x_ref, y_ref, z_ref, out_ref, dists_ref):
    nchunks = N // CHUNK
    dists_ref[...] = jnp.full((B, N), 1e10, dtype=jnp.float32)
    outpos = lax.broadcasted_iota(jnp.int32, (B, NPOINT), 1)
    chunk_lane = lax.broadcasted_iota(jnp.int32, (B, CHUNK), 1)
    out_ref[...] = jnp.zeros((B, NPOINT), jnp.int32)

    def it(i, carry):
        f, cx, cy, cz = carry
        out_ref[...] = jnp.where(outpos == i, f, out_ref[...])

        mv = jnp.full((B, CHUNK), -1.0, jnp.float32)
        iv = jnp.zeros((B, CHUNK), jnp.int32)
        xv = jnp.zeros((B, CHUNK), jnp.float32)
        yv = jnp.zeros((B, CHUNK), jnp.float32)
        zv = jnp.zeros((B, CHUNK), jnp.float32)

        for c in range(nchunks):
            sl = pl.ds(c * CHUNK, CHUNK)
            xc = x_ref[:, sl]
            yc = y_ref[:, sl]
            zc = z_ref[:, sl]
            dx = xc - cx
            dy = yc - cy
            dz = zc - cz
            d = (dx * dx + dz * dz) + dy * dy
            nd = jnp.minimum(dists_ref[:, sl], d)
            dists_ref[:, sl] = nd
            m = nd > mv
            mv = jnp.where(m, nd, mv)
            iv = jnp.where(m, chunk_lane + (c * CHUNK), iv)
            xv = jnp.where(m, xc, xv)
            yv = jnp.where(m, yc, yv)
            zv = jnp.where(m, zc, zv)

        gmax = jnp.max(mv, axis=1, keepdims=True)
        nf = jnp.min(jnp.where(mv == gmax, iv, N), axis=1, keepdims=True)
        win = iv == nf
        ncx = jnp.sum(jnp.where(win, xv, 0.0), axis=1, keepdims=True)
        ncy = jnp.sum(jnp.where(win, yv, 0.0), axis=1, keepdims=True)
        ncz = jnp.sum(jnp.where(win, zv, 0.0), axis=1, keepdims=True)
        return nf, ncx, ncy, ncz

    f0 = jnp.zeros((B, 1), jnp.int32)
    lax.fori_loop(
        0, NPOINT, it,
        (f0, x_ref[:, 0:1], y_ref[:, 0:1], z_ref[:, 0:1]),
    )


def kernel(pts):
    ptsT = jnp.transpose(pts, (2, 0, 1))
    x, y, z = ptsT[0], ptsT[1], ptsT[2]
    spec = pl.BlockSpec((B, N), lambda: (0, 0))
    out = pl.pallas_call(
        _fps_body,
        in_specs=[spec, spec, spec],
        out_specs=pl.BlockSpec((B, NPOINT), lambda: (0, 0)),
        out_shape=jax.ShapeDtypeStruct((B, NPOINT), jnp.int32),
        scratch_shapes=[pltpu.VMEM((B, N), jnp.float32)],
    )(x, y, z)
    return out
